# baseline (speedup 1.0000x reference)
import functools

import jax
import jax.numpy as jnp
from jax import lax
from jax.experimental import pallas as pl
from jax.experimental.pallas import tpu as pltpu
from jax.experimental.pallas import tpu_sc as plsc

N = 10000
D = 128
E = 160000
DAUG = 144
K = 128
NS = 16
CH = 80
EPT = CH * K
NA = N + 8
QC = 4
ZR = 80
NZ = N // ZR
BM = 1000


def _aug_ones(bm):
    return (lax.broadcasted_iota(jnp.int32, (bm, DAUG - D), 1) == 0).astype(
        jnp.float32)


def _tc1_body(xp, xa, wsp, wpa, wsa, wap, sp, sa, tap, tpa):
    xpv = xp[...]
    xav = xa[...]
    sp[...] = jnp.dot(xpv, wsp[...], preferred_element_type=jnp.float32)
    sa[...] = jnp.dot(xav, wsa[...], preferred_element_type=jnp.float32)
    aug = _aug_ones(xpv.shape[0])
    tap[...] = jnp.concatenate(
        [jnp.dot(xav, wap[...], preferred_element_type=jnp.float32), aug],
        axis=1)
    tpa[...] = jnp.concatenate(
        [jnp.dot(xpv, wpa[...], preferred_element_type=jnp.float32), aug],
        axis=1)


def _tc2_body(aggp, agga, sp0, sa0, wsp, wpa, wsa, wap, sp1, sa1, tap, tpa):
    ap = aggp[...]
    aa = agga[...]
    hp = jax.nn.relu(sp0[...] + ap[:, :D] / jnp.clip(ap[:, D:D + 1], 1.0))
    ha = jax.nn.relu(sa0[...] + aa[:, :D] / jnp.clip(aa[:, D:D + 1], 1.0))
    sp1[...] = jnp.dot(hp, wsp[...], preferred_element_type=jnp.float32)
    sa1[...] = jnp.dot(ha, wsa[...], preferred_element_type=jnp.float32)
    tap[...] = jnp.dot(ha, wap[...], preferred_element_type=jnp.float32)
    tpa[...] = jnp.dot(hp, wpa[...], preferred_element_type=jnp.float32)


def _tc3_body(aggp, agga, dgp, dga, sp1, sa1, zp, za):
    zp[...] = sp1[...] + aggp[...] / jnp.clip(dgp[:, 0:1], 1.0)
    za[...] = sa1[...] + agga[...] / jnp.clip(dga[:, 0:1], 1.0)


_bs_x = pl.BlockSpec((BM, D), lambda i: (i, 0))
_bs_w = pl.BlockSpec((D, D), lambda i: (0, 0))
_bs_d = pl.BlockSpec((BM, DAUG - D), lambda i: (i, 0))
_bs_aug = pl.BlockSpec((BM, DAUG), lambda i: (i, 0))
_sds_x = jax.ShapeDtypeStruct((N, D), jnp.float32)
_sds_aug = jax.ShapeDtypeStruct((N, DAUG), jnp.float32)

_tc1 = pl.pallas_call(
    _tc1_body,
    grid=(N // BM,),
    in_specs=[_bs_x, _bs_x, _bs_w, _bs_w, _bs_w, _bs_w],
    out_specs=[_bs_x, _bs_x, _bs_aug, _bs_aug],
    out_shape=[_sds_x, _sds_x, _sds_aug, _sds_aug],
)

_tc2 = pl.pallas_call(
    _tc2_body,
    grid=(N // BM,),
    in_specs=[_bs_aug, _bs_aug, _bs_x, _bs_x, _bs_w, _bs_w, _bs_w, _bs_w],
    out_specs=[_bs_x, _bs_x, _bs_x, _bs_x],
    out_shape=[_sds_x, _sds_x, _sds_x, _sds_x],
)

_tc3 = pl.pallas_call(
    _tc3_body,
    grid=(N // BM,),
    in_specs=[_bs_x, _bs_x, _bs_d, _bs_d, _bs_x, _bs_x],
    out_specs=[_bs_x, _bs_x],
    out_shape=[_sds_x, _sds_x],
)


@functools.cache
def _make_sc_agg(width):
    mesh = plsc.VectorSubcoreMesh(core_axis_name="c", subcore_axis_name="s")
    return pl.kernel(
        functools.partial(_sc_agg_body, width),
        out_type=[jax.ShapeDtypeStruct((N, width), jnp.float32),
                  jax.ShapeDtypeStruct((N, width), jnp.float32)],
        mesh=mesh,
        scratch_types=[
            pltpu.VMEM((QC * K,), jnp.int32),
            pltpu.VMEM((QC * K,), jnp.int32),
            pltpu.VMEM((QC * K,), jnp.int32),
            pltpu.VMEM((QC * K,), jnp.int32),
            pltpu.VMEM((K, width), jnp.float32),
            pltpu.VMEM((K, width), jnp.float32),
            pltpu.VMEM_SHARED((NA, width), jnp.float32),
            pltpu.SemaphoreType.DMA,
            pltpu.SemaphoreType.DMA,
            pltpu.SemaphoreType.DMA,
            pltpu.SemaphoreType.DMA,
        ],
        compiler_params=pltpu.CompilerParams(use_tc_tiling_on_sc=False),
    )


def _sc_agg_body(width, tap, tpa, src_ap, dst_ap, src_pa, dst_pa, zrows,
                 out_p, out_a, qs0, qd0, qs1, qd1, rows0, rows1, acc,
                 iqsem0, iqsem1, gsem0, gsem1):
    cid = lax.axis_index("c")
    sid = lax.axis_index("s")
    QK = QC * K

    def run(table, srcf, dstf, out):
        for k in range(pl.cdiv(NZ, NS)):
            j = sid + k * NS

            @pl.when(j < NZ)
            def _():
                pltpu.sync_copy(zrows, acc.at[pl.ds(j * ZR, ZR)])

        plsc.subcore_barrier()

        ebase = sid * EPT

        def fetch_idx(off, qs, qd, sem):
            pltpu.async_copy(srcf.at[pl.ds(ebase + off * K, QK)], qs, sem)
            pltpu.async_copy(dstf.at[pl.ds(ebase + off * K, QK)], qd, sem)

        def wait_idx(off, qs, qd, sem):
            pltpu.make_async_copy(srcf.at[pl.ds(ebase + off * K, QK)], qs,
                                  sem).wait()
            pltpu.make_async_copy(dstf.at[pl.ds(ebase + off * K, QK)], qd,
                                  sem).wait()

        fetch_idx(0, qs0, qd0, iqsem0)
        wait_idx(0, qs0, qd0, iqsem0)
        pltpu.async_copy(table.at[qs0.at[pl.ds(0, K)]], rows0, gsem0)
        pltpu.async_copy(table.at[qs0.at[pl.ds(K, K)]], rows1, gsem1)
        fetch_idx(QC, qs1, qd1, iqsem1)

        @pl.loop(0, CH, step=2 * QC)
        def _(j):
            for m in range(2 * QC):
                rows_m = rows0 if m % 2 == 0 else rows1
                gsem_m = gsem0 if m % 2 == 0 else gsem1
                qs_m = qs0 if m < QC else qs1
                qd_m = qd0 if m < QC else qd1
                o_m = (m % QC) * K
                pltpu.make_async_copy(table.at[qs_m.at[pl.ds(o_m, K)]],
                                      rows_m, gsem_m).wait()
                pltpu.sync_copy(rows_m, acc.at[qd_m.at[pl.ds(o_m, K)]],
                                add=True)

                if m == QC - 2:
                    wait_idx(j + QC, qs1, qd1, iqsem1)

                if m == QC - 1:
                    @pl.when(j + 2 * QC < CH)
                    def _():
                        fetch_idx(j + 2 * QC, qs0, qd0, iqsem0)

                c = m + 2
                if c < QC:
                    pltpu.async_copy(table.at[qs0.at[pl.ds(c * K, K)]],
                                     rows_m, gsem_m)
                elif c < 2 * QC:
                    pltpu.async_copy(
                        table.at[qs1.at[pl.ds((c - QC) * K, K)]], rows_m,
                        gsem_m)
                else:
                    if c == 2 * QC:
                        @pl.when(j + 2 * QC < CH)
                        def _():
                            wait_idx(j + 2 * QC, qs0, qd0, iqsem0)

                    @pl.when(j + c < CH)
                    def _():
                        pltpu.async_copy(
                            table.at[qs0.at[pl.ds((c - 2 * QC) * K, K)]],
                            rows_m, gsem_m)

                if m == 2 * QC - 1:
                    @pl.when(j + 3 * QC < CH)
                    def _():
                        fetch_idx(j + 3 * QC, qs1, qd1, iqsem1)

        plsc.subcore_barrier()
        for k in range(pl.cdiv(NZ, NS)):
            j = sid + k * NS

            @pl.when(j < NZ)
            def _():
                pltpu.sync_copy(acc.at[pl.ds(j * ZR, ZR)],
                                out.at[pl.ds(j * ZR, ZR)])

    @pl.when(cid == 0)
    def _():
        run(tap, src_ap, dst_ap, out_p)

    @pl.when(cid == 1)
    def _():
        run(tpa, src_pa, dst_pa, out_a)


def kernel(x_paper, x_author, edge_index_ap, edge_index_pa,
           W_ap_0, W_pa_0, W_sp_0, W_sa_0,
           W_ap_1, W_pa_1, W_sp_1, W_sa_1):
    def pack_idx(ei):
        e3 = ei.astype(jnp.int32).reshape(2, NS, E // NS)
        pad = jnp.broadcast_to(
            jnp.array([0, N], jnp.int32)[:, None, None],
            (2, NS, EPT - E // NS))
        e3 = jnp.concatenate([e3, pad], axis=2)
        return e3.reshape(2, NS * EPT)

    src_ap, dst_ap = pack_idx(edge_index_ap)
    src_pa, dst_pa = pack_idx(edge_index_pa)
    zrows_aug = jnp.zeros((ZR, DAUG), jnp.float32)
    zrows = jnp.zeros((ZR, D), jnp.float32)

    sc_agg1 = _make_sc_agg(DAUG)
    sc_agg2 = _make_sc_agg(D)
    sp0, sa0, tap0, tpa0 = _tc1(x_paper, x_author, W_sp_0, W_pa_0, W_sa_0,
                                W_ap_0)
    aggp0, agga0 = sc_agg1(tap0, tpa0, src_ap, dst_ap, src_pa, dst_pa,
                           zrows_aug)
    sp1, sa1, tap1, tpa1 = _tc2(aggp0, agga0, sp0, sa0,
                                W_sp_1, W_pa_1, W_sa_1, W_ap_1)
    aggp1, agga1 = sc_agg2(tap1, tpa1, src_ap, dst_ap, src_pa, dst_pa, zrows)
    dgp = lax.slice(aggp0, (0, D), (N, DAUG))
    dga = lax.slice(agga0, (0, D), (N, DAUG))
    zp, za = _tc3(aggp1, agga1, dgp, dga, sp1, sa1)
    return jnp.concatenate([zp, za], axis=0)


# speedup vs baseline: 1.7118x; 1.7118x over previous
import functools

import jax
import jax.numpy as jnp
from jax import lax
from jax.experimental import pallas as pl
from jax.experimental.pallas import tpu as pltpu
from jax.experimental.pallas import tpu_sc as plsc

N = 10000
D = 128
E = 160000
DAUG = 144
K = 125
NS = 16
EPT = E // NS
CH = EPT // K
QC = 8
ZR = 80
NZ = N // ZR
BM = 1000


def _aug_ones(bm):
    return (lax.broadcasted_iota(jnp.int32, (bm, DAUG - D), 1) == 0).astype(
        jnp.float32)


def _tc1_body(xp, xa, wsp, wpa, wsa, wap, sp, sa, tap, tpa):
    xpv = xp[...]
    xav = xa[...]
    sp[...] = jnp.dot(xpv, wsp[...], preferred_element_type=jnp.float32)
    sa[...] = jnp.dot(xav, wsa[...], preferred_element_type=jnp.float32)
    aug = _aug_ones(xpv.shape[0])
    tap[...] = jnp.concatenate(
        [jnp.dot(xav, wap[...], preferred_element_type=jnp.float32), aug],
        axis=1)
    tpa[...] = jnp.concatenate(
        [jnp.dot(xpv, wpa[...], preferred_element_type=jnp.float32), aug],
        axis=1)


def _tc2_body(aggp, agga, sp0, sa0, wsp, wpa, wsa, wap, sp1, sa1, tap, tpa):
    ap = aggp[...]
    aa = agga[...]
    hp = jax.nn.relu(sp0[...] + ap[:, :D] / jnp.clip(ap[:, D:D + 1], 1.0))
    ha = jax.nn.relu(sa0[...] + aa[:, :D] / jnp.clip(aa[:, D:D + 1], 1.0))
    sp1[...] = jnp.dot(hp, wsp[...], preferred_element_type=jnp.float32)
    sa1[...] = jnp.dot(ha, wsa[...], preferred_element_type=jnp.float32)
    tap[...] = jnp.dot(ha, wap[...], preferred_element_type=jnp.float32)
    tpa[...] = jnp.dot(hp, wpa[...], preferred_element_type=jnp.float32)


def _tc3_body(aggp, agga, dgp, dga, sp1, sa1, zp, za):
    zp[...] = sp1[...] + aggp[...] / jnp.clip(dgp[:, 0:1], 1.0)
    za[...] = sa1[...] + agga[...] / jnp.clip(dga[:, 0:1], 1.0)


_bs_x = pl.BlockSpec((BM, D), lambda i: (i, 0))
_bs_w = pl.BlockSpec((D, D), lambda i: (0, 0))
_bs_d = pl.BlockSpec((BM, DAUG - D), lambda i: (i, 0))
_bs_aug = pl.BlockSpec((BM, DAUG), lambda i: (i, 0))
_sds_x = jax.ShapeDtypeStruct((N, D), jnp.float32)
_sds_aug = jax.ShapeDtypeStruct((N, DAUG), jnp.float32)

_tc1 = pl.pallas_call(
    _tc1_body,
    grid=(N // BM,),
    in_specs=[_bs_x, _bs_x, _bs_w, _bs_w, _bs_w, _bs_w],
    out_specs=[_bs_x, _bs_x, _bs_aug, _bs_aug],
    out_shape=[_sds_x, _sds_x, _sds_aug, _sds_aug],
)

_tc2 = pl.pallas_call(
    _tc2_body,
    grid=(N // BM,),
    in_specs=[_bs_aug, _bs_aug, _bs_x, _bs_x, _bs_w, _bs_w, _bs_w, _bs_w],
    out_specs=[_bs_x, _bs_x, _bs_x, _bs_x],
    out_shape=[_sds_x, _sds_x, _sds_x, _sds_x],
)

_tc3 = pl.pallas_call(
    _tc3_body,
    grid=(N // BM,),
    in_specs=[_bs_x, _bs_x, _bs_d, _bs_d, _bs_x, _bs_x],
    out_specs=[_bs_x, _bs_x],
    out_shape=[_sds_x, _sds_x],
)


@functools.cache
def _make_sc_agg(width):
    mesh = plsc.VectorSubcoreMesh(core_axis_name="c", subcore_axis_name="s")
    return pl.kernel(
        functools.partial(_sc_agg_body, width),
        out_type=[jax.ShapeDtypeStruct((N, width), jnp.float32),
                  jax.ShapeDtypeStruct((N, width), jnp.float32)],
        mesh=mesh,
        scratch_types=[
            pltpu.VMEM((QC, 2, K), jnp.int32),
            pltpu.VMEM((QC, 2, K), jnp.int32),
            pltpu.VMEM((K, width), jnp.float32),
            pltpu.VMEM((K, width), jnp.float32),
            pltpu.VMEM_SHARED((N, width), jnp.float32),
            pltpu.SemaphoreType.DMA,
            pltpu.SemaphoreType.DMA,
            pltpu.SemaphoreType.DMA,
            pltpu.SemaphoreType.DMA,
        ],
        compiler_params=pltpu.CompilerParams(use_tc_tiling_on_sc=False),
    )


def _sc_agg_body(width, tap, tpa, idx_ap, idx_pa, zrows,
                 out_p, out_a, q0, q1, rows0, rows1, acc,
                 iqsem0, iqsem1, gsem0, gsem1):
    cid = lax.axis_index("c")
    sid = lax.axis_index("s")

    def run(table, idx3d, out):
        for k in range(pl.cdiv(NZ, NS)):
            j = sid + k * NS

            @pl.when(j < NZ)
            def _():
                pltpu.sync_copy(zrows, acc.at[pl.ds(j * ZR, ZR)])

        plsc.subcore_barrier()

        base = sid * CH
        pltpu.sync_copy(idx3d.at[pl.ds(base, QC)], q0)
        pltpu.async_copy(table.at[q0.at[0, 0]], rows0, gsem0)
        pltpu.async_copy(table.at[q0.at[1, 0]], rows1, gsem1)
        pltpu.async_copy(idx3d.at[pl.ds(base + QC, QC)], q1, iqsem1)

        @pl.loop(0, CH, step=2 * QC)
        def _(j):
            for m in range(2 * QC):
                rows_m = rows0 if m % 2 == 0 else rows1
                gsem_m = gsem0 if m % 2 == 0 else gsem1
                q_m = q0 if m < QC else q1
                pltpu.make_async_copy(table.at[q_m.at[m % QC, 0]], rows_m,
                                      gsem_m).wait()
                pltpu.sync_copy(rows_m, acc.at[q_m.at[m % QC, 1]], add=True)

                if m == QC - 2:
                    pltpu.make_async_copy(
                        idx3d.at[pl.ds(base + j + QC, QC)], q1, iqsem1).wait()

                if m == QC - 1:
                    @pl.when(j + 2 * QC < CH)
                    def _():
                        pltpu.async_copy(
                            idx3d.at[pl.ds(base + j + 2 * QC, QC)], q0,
                            iqsem0)

                c = m + 2
                if c < QC:
                    pltpu.async_copy(table.at[q0.at[c, 0]], rows_m, gsem_m)
                elif c < 2 * QC:
                    pltpu.async_copy(table.at[q1.at[c - QC, 0]], rows_m,
                                     gsem_m)
                else:
                    if c == 2 * QC:
                        @pl.when(j + 2 * QC < CH)
                        def _():
                            pltpu.make_async_copy(
                                idx3d.at[pl.ds(base + j + 2 * QC, QC)], q0,
                                iqsem0).wait()

                    @pl.when(j + c < CH)
                    def _():
                        pltpu.async_copy(table.at[q0.at[c - 2 * QC, 0]],
                                         rows_m, gsem_m)

                if m == 2 * QC - 1:
                    @pl.when(j + 3 * QC < CH)
                    def _():
                        pltpu.async_copy(
                            idx3d.at[pl.ds(base + j + 3 * QC, QC)], q1,
                            iqsem1)

        plsc.subcore_barrier()
        for k in range(pl.cdiv(NZ, NS)):
            j = sid + k * NS

            @pl.when(j < NZ)
            def _():
                pltpu.sync_copy(acc.at[pl.ds(j * ZR, ZR)],
                                out.at[pl.ds(j * ZR, ZR)])

    @pl.when(cid == 0)
    def _():
        run(tap, idx_ap, out_p)

    @pl.when(cid == 1)
    def _():
        run(tpa, idx_pa, out_a)


def kernel(x_paper, x_author, edge_index_ap, edge_index_pa,
           W_ap_0, W_pa_0, W_sp_0, W_sa_0,
           W_ap_1, W_pa_1, W_sp_1, W_sa_1):
    eap = edge_index_ap.astype(jnp.int32)
    epa = edge_index_pa.astype(jnp.int32)
    idx_ap = eap.reshape(2, E // K, K).transpose(1, 0, 2)
    idx_pa = epa.reshape(2, E // K, K).transpose(1, 0, 2)
    zrows_aug = jnp.zeros((ZR, DAUG), jnp.float32)
    zrows = jnp.zeros((ZR, D), jnp.float32)

    sc_agg1 = _make_sc_agg(DAUG)
    sc_agg2 = _make_sc_agg(D)
    sp0, sa0, tap0, tpa0 = _tc1(x_paper, x_author, W_sp_0, W_pa_0, W_sa_0,
                                W_ap_0)
    aggp0, agga0 = sc_agg1(tap0, tpa0, idx_ap, idx_pa, zrows_aug)
    sp1, sa1, tap1, tpa1 = _tc2(aggp0, agga0, sp0, sa0,
                                W_sp_1, W_pa_1, W_sa_1, W_ap_1)
    aggp1, agga1 = sc_agg2(tap1, tpa1, idx_ap, idx_pa, zrows)
    dgp = lax.slice(aggp0, (0, D), (N, DAUG))
    dga = lax.slice(agga0, (0, D), (N, DAUG))
    zp, za = _tc3(aggp1, agga1, dgp, dga, sp1, sa1)
    return jnp.concatenate([zp, za], axis=0)


# speedup vs baseline: 1.7608x; 1.0286x over previous
import functools

import jax
import jax.numpy as jnp
from jax import lax
from jax.experimental import pallas as pl
from jax.experimental.pallas import tpu as pltpu
from jax.experimental.pallas import tpu_sc as plsc

N = 10000
D = 128
E = 160000
DAUG = 144
K = 125
NS = 16
EPT = E // NS
CH = EPT // K
QC = 8
ZR = 80
NZ = N // ZR
BM = 1000


def _aug_ones(bm):
    return (lax.broadcasted_iota(jnp.int32, (bm, DAUG - D), 1) == 0).astype(
        jnp.float32)


def _tc1_body(xp, xa, wsp, wpa, wsa, wap, sp, sa, tap, tpa):
    xpv = xp[...]
    xav = xa[...]
    sp[...] = jnp.dot(xpv, wsp[...], preferred_element_type=jnp.float32)
    sa[...] = jnp.dot(xav, wsa[...], preferred_element_type=jnp.float32)
    aug = _aug_ones(xpv.shape[0])
    tap[...] = jnp.concatenate(
        [jnp.dot(xav, wap[...], preferred_element_type=jnp.float32), aug],
        axis=1)
    tpa[...] = jnp.concatenate(
        [jnp.dot(xpv, wpa[...], preferred_element_type=jnp.float32), aug],
        axis=1)


def _tc2_body(aggp, agga, sp0, sa0, wsp, wpa, wsa, wap, sp1, sa1, tap, tpa):
    ap = aggp[...]
    aa = agga[...]
    hp = jax.nn.relu(sp0[...] + ap[:, :D] / jnp.clip(ap[:, D:D + 1], 1.0))
    ha = jax.nn.relu(sa0[...] + aa[:, :D] / jnp.clip(aa[:, D:D + 1], 1.0))
    sp1[...] = jnp.dot(hp, wsp[...], preferred_element_type=jnp.float32)
    sa1[...] = jnp.dot(ha, wsa[...], preferred_element_type=jnp.float32)
    tap[...] = jnp.dot(ha, wap[...], preferred_element_type=jnp.float32)
    tpa[...] = jnp.dot(hp, wpa[...], preferred_element_type=jnp.float32)


def _tc3_body(aggp, agga, dgp, dga, sp1, sa1, zp, za):
    zp[...] = sp1[...] + aggp[...] / jnp.clip(dgp[:, 0:1], 1.0)
    za[...] = sa1[...] + agga[...] / jnp.clip(dga[:, 0:1], 1.0)


_bs_x = pl.BlockSpec((BM, D), lambda i: (i, 0))
_bs_w = pl.BlockSpec((D, D), lambda i: (0, 0))
_bs_d = pl.BlockSpec((BM, DAUG - D), lambda i: (i, 0))
_bs_aug = pl.BlockSpec((BM, DAUG), lambda i: (i, 0))
_sds_x = jax.ShapeDtypeStruct((N, D), jnp.float32)
_sds_aug = jax.ShapeDtypeStruct((N, DAUG), jnp.float32)

_tc1 = pl.pallas_call(
    _tc1_body,
    grid=(N // BM,),
    in_specs=[_bs_x, _bs_x, _bs_w, _bs_w, _bs_w, _bs_w],
    out_specs=[_bs_x, _bs_x, _bs_aug, _bs_aug],
    out_shape=[_sds_x, _sds_x, _sds_aug, _sds_aug],
)

_tc2 = pl.pallas_call(
    _tc2_body,
    grid=(N // BM,),
    in_specs=[_bs_aug, _bs_aug, _bs_x, _bs_x, _bs_w, _bs_w, _bs_w, _bs_w],
    out_specs=[_bs_x, _bs_x, _bs_x, _bs_x],
    out_shape=[_sds_x, _sds_x, _sds_x, _sds_x],
)

_tc3 = pl.pallas_call(
    _tc3_body,
    grid=(N // BM,),
    in_specs=[_bs_x, _bs_x, _bs_d, _bs_d, _bs_x, _bs_x],
    out_specs=[_bs_x, _bs_x],
    out_shape=[_sds_x, _sds_x],
)


@functools.cache
def _make_sc_agg(width):
    mesh = plsc.VectorSubcoreMesh(core_axis_name="c", subcore_axis_name="s")
    return pl.kernel(
        functools.partial(_sc_agg_body, width),
        out_type=[jax.ShapeDtypeStruct((N, width), jnp.float32),
                  jax.ShapeDtypeStruct((N, width), jnp.float32)],
        mesh=mesh,
        scratch_types=[
            pltpu.VMEM((QC, K), jnp.int32),
            pltpu.VMEM((QC, K), jnp.int32),
            pltpu.VMEM((QC, K), jnp.int32),
            pltpu.VMEM((QC, K), jnp.int32),
            pltpu.VMEM((K, width), jnp.float32),
            pltpu.VMEM((K, width), jnp.float32),
            pltpu.VMEM_SHARED((N, width), jnp.float32),
            pltpu.SemaphoreType.DMA,
            pltpu.SemaphoreType.DMA,
            pltpu.SemaphoreType.DMA,
            pltpu.SemaphoreType.DMA,
        ],
        compiler_params=pltpu.CompilerParams(use_tc_tiling_on_sc=False),
    )


def _sc_agg_body(width, tap, tpa, idx_ap, idx_pa, zrows,
                 out_p, out_a, qs0, qd0, qs1, qd1, rows0, rows1, acc,
                 iqsem0, iqsem1, gsem0, gsem1):
    cid = lax.axis_index("c")
    sid = lax.axis_index("s")

    def run(table, idx3d, out):
        for k in range(pl.cdiv(NZ, NS)):
            j = sid + k * NS

            @pl.when(j < NZ)
            def _():
                pltpu.sync_copy(zrows, acc.at[pl.ds(j * ZR, ZR)])

        plsc.subcore_barrier()

        base = sid * CH

        def fetch_idx(off, qs, qd, sem):
            pltpu.async_copy(idx3d.at[0, pl.ds(base + off, QC)], qs, sem)
            pltpu.async_copy(idx3d.at[1, pl.ds(base + off, QC)], qd, sem)

        def wait_idx(off, qs, qd, sem):
            pltpu.make_async_copy(idx3d.at[0, pl.ds(base + off, QC)], qs,
                                  sem).wait()
            pltpu.make_async_copy(idx3d.at[1, pl.ds(base + off, QC)], qd,
                                  sem).wait()

        fetch_idx(0, qs0, qd0, iqsem0)
        wait_idx(0, qs0, qd0, iqsem0)
        pltpu.async_copy(table.at[qs0.at[0]], rows0, gsem0)
        pltpu.async_copy(table.at[qs0.at[1]], rows1, gsem1)
        fetch_idx(QC, qs1, qd1, iqsem1)

        @pl.loop(0, CH, step=2 * QC)
        def _(j):
            for m in range(2 * QC):
                rows_m = rows0 if m % 2 == 0 else rows1
                gsem_m = gsem0 if m % 2 == 0 else gsem1
                qs_m = qs0 if m < QC else qs1
                qd_m = qd0 if m < QC else qd1
                pltpu.make_async_copy(table.at[qs_m.at[m % QC]], rows_m,
                                      gsem_m).wait()
                pltpu.sync_copy(rows_m, acc.at[qd_m.at[m % QC]], add=True)

                if m == QC - 2:
                    wait_idx(j + QC, qs1, qd1, iqsem1)

                if m == QC - 1:
                    @pl.when(j + 2 * QC < CH)
                    def _():
                        fetch_idx(j + 2 * QC, qs0, qd0, iqsem0)

                c = m + 2
                if c < QC:
                    pltpu.async_copy(table.at[qs0.at[c]], rows_m, gsem_m)
                elif c < 2 * QC:
                    pltpu.async_copy(table.at[qs1.at[c - QC]], rows_m,
                                     gsem_m)
                else:
                    if c == 2 * QC:
                        @pl.when(j + 2 * QC < CH)
                        def _():
                            wait_idx(j + 2 * QC, qs0, qd0, iqsem0)

                    @pl.when(j + c < CH)
                    def _():
                        pltpu.async_copy(table.at[qs0.at[c - 2 * QC]],
                                         rows_m, gsem_m)

                if m == 2 * QC - 1:
                    @pl.when(j + 3 * QC < CH)
                    def _():
                        fetch_idx(j + 3 * QC, qs1, qd1, iqsem1)

        plsc.subcore_barrier()
        for k in range(pl.cdiv(NZ, NS)):
            j = sid + k * NS

            @pl.when(j < NZ)
            def _():
                pltpu.sync_copy(acc.at[pl.ds(j * ZR, ZR)],
                                out.at[pl.ds(j * ZR, ZR)])

    @pl.when(cid == 0)
    def _():
        run(tap, idx_ap, out_p)

    @pl.when(cid == 1)
    def _():
        run(tpa, idx_pa, out_a)


def kernel(x_paper, x_author, edge_index_ap, edge_index_pa,
           W_ap_0, W_pa_0, W_sp_0, W_sa_0,
           W_ap_1, W_pa_1, W_sp_1, W_sa_1):
    idx_ap = edge_index_ap.astype(jnp.int32).reshape(2, E // K, K)
    idx_pa = edge_index_pa.astype(jnp.int32).reshape(2, E // K, K)
    zrows_aug = jnp.zeros((ZR, DAUG), jnp.float32)
    zrows = jnp.zeros((ZR, D), jnp.float32)

    sc_agg1 = _make_sc_agg(DAUG)
    sc_agg2 = _make_sc_agg(D)
    sp0, sa0, tap0, tpa0 = _tc1(x_paper, x_author, W_sp_0, W_pa_0, W_sa_0,
                                W_ap_0)
    aggp0, agga0 = sc_agg1(tap0, tpa0, idx_ap, idx_pa, zrows_aug)
    sp1, sa1, tap1, tpa1 = _tc2(aggp0, agga0, sp0, sa0,
                                W_sp_1, W_pa_1, W_sa_1, W_ap_1)
    aggp1, agga1 = sc_agg2(tap1, tpa1, idx_ap, idx_pa, zrows)
    dgp = lax.slice(aggp0, (0, D), (N, DAUG))
    dga = lax.slice(agga0, (0, D), (N, DAUG))
    zp, za = _tc3(aggp1, agga1, dgp, dga, sp1, sa1)
    return jnp.concatenate([zp, za], axis=0)


# speedup vs baseline: 1.7631x; 1.0013x over previous
import functools

import jax
import jax.numpy as jnp
from jax import lax
from jax.experimental import pallas as pl
from jax.experimental.pallas import tpu as pltpu
from jax.experimental.pallas import tpu_sc as plsc

N = 10000
D = 128
E = 160000
DAUG = 144
K = 125
NS = 16
EPT = E // NS
CH = EPT // K
QC = 8
ZR = 80
NZ = N // ZR
BM = 1000


def _aug_ones(bm):
    return (lax.broadcasted_iota(jnp.int32, (bm, DAUG - D), 1) == 0).astype(
        jnp.float32)


def _tc1_body(xp, xa, wsp, wpa, wsa, wap, sp, sa, tap, tpa):
    xpv = xp[...]
    xav = xa[...]
    sp[...] = jnp.dot(xpv, wsp[...], preferred_element_type=jnp.float32)
    sa[...] = jnp.dot(xav, wsa[...], preferred_element_type=jnp.float32)
    aug = _aug_ones(xpv.shape[0])
    tap[...] = jnp.concatenate(
        [jnp.dot(xav, wap[...], preferred_element_type=jnp.float32), aug],
        axis=1)
    tpa[...] = jnp.concatenate(
        [jnp.dot(xpv, wpa[...], preferred_element_type=jnp.float32), aug],
        axis=1)


def _tc2_body(aggp, agga, sp0, sa0, wsp, wpa, wsa, wap, sp1, sa1, tap, tpa):
    ap = aggp[...]
    aa = agga[...]
    hp = jax.nn.relu(sp0[...] + ap[:, :D] / jnp.clip(ap[:, D:D + 1], 1.0))
    ha = jax.nn.relu(sa0[...] + aa[:, :D] / jnp.clip(aa[:, D:D + 1], 1.0))
    sp1[...] = jnp.dot(hp, wsp[...], preferred_element_type=jnp.float32)
    sa1[...] = jnp.dot(ha, wsa[...], preferred_element_type=jnp.float32)
    tap[...] = jnp.dot(ha, wap[...], preferred_element_type=jnp.float32)
    tpa[...] = jnp.dot(hp, wpa[...], preferred_element_type=jnp.float32)


def _tc3_body(aggp, agga, ag0p, ag0a, sp1, sa1, zp, za):
    zp[...] = sp1[...] + aggp[...] / jnp.clip(ag0p[:, D:D + 1], 1.0)
    za[...] = sa1[...] + agga[...] / jnp.clip(ag0a[:, D:D + 1], 1.0)


_bs_x = pl.BlockSpec((BM, D), lambda i: (i, 0))
_bs_w = pl.BlockSpec((D, D), lambda i: (0, 0))
_bs_aug = pl.BlockSpec((BM, DAUG), lambda i: (i, 0))
_sds_x = jax.ShapeDtypeStruct((N, D), jnp.float32)
_sds_aug = jax.ShapeDtypeStruct((N, DAUG), jnp.float32)

_tc1 = pl.pallas_call(
    _tc1_body,
    grid=(N // BM,),
    in_specs=[_bs_x, _bs_x, _bs_w, _bs_w, _bs_w, _bs_w],
    out_specs=[_bs_x, _bs_x, _bs_aug, _bs_aug],
    out_shape=[_sds_x, _sds_x, _sds_aug, _sds_aug],
)

_tc2 = pl.pallas_call(
    _tc2_body,
    grid=(N // BM,),
    in_specs=[_bs_aug, _bs_aug, _bs_x, _bs_x, _bs_w, _bs_w, _bs_w, _bs_w],
    out_specs=[_bs_x, _bs_x, _bs_x, _bs_x],
    out_shape=[_sds_x, _sds_x, _sds_x, _sds_x],
)

_tc3 = pl.pallas_call(
    _tc3_body,
    grid=(N // BM,),
    in_specs=[_bs_x, _bs_x, _bs_aug, _bs_aug, _bs_x, _bs_x],
    out_specs=[_bs_x, _bs_x],
    out_shape=[_sds_x, _sds_x],
)


@functools.cache
def _make_sc_agg(width):
    mesh = plsc.VectorSubcoreMesh(core_axis_name="c", subcore_axis_name="s")
    return pl.kernel(
        functools.partial(_sc_agg_body, width),
        out_type=[jax.ShapeDtypeStruct((N, width), jnp.float32),
                  jax.ShapeDtypeStruct((N, width), jnp.float32)],
        mesh=mesh,
        scratch_types=[
            pltpu.VMEM((QC, K), jnp.int32),
            pltpu.VMEM((QC, K), jnp.int32),
            pltpu.VMEM((QC, K), jnp.int32),
            pltpu.VMEM((QC, K), jnp.int32),
            pltpu.VMEM((K, width), jnp.float32),
            pltpu.VMEM((K, width), jnp.float32),
            pltpu.VMEM_SHARED((N, width), jnp.float32),
            pltpu.SemaphoreType.DMA,
            pltpu.SemaphoreType.DMA,
            pltpu.SemaphoreType.DMA,
            pltpu.SemaphoreType.DMA,
        ],
        compiler_params=pltpu.CompilerParams(use_tc_tiling_on_sc=False),
    )


def _sc_agg_body(width, tap, tpa, idx_ap, idx_pa, zrows,
                 out_p, out_a, qs0, qd0, qs1, qd1, rows0, rows1, acc,
                 iqsem0, iqsem1, gsem0, gsem1):
    cid = lax.axis_index("c")
    sid = lax.axis_index("s")

    def run(table, idx3d, out):
        for k in range(pl.cdiv(NZ, NS)):
            j = sid + k * NS

            @pl.when(j < NZ)
            def _():
                pltpu.sync_copy(zrows, acc.at[pl.ds(j * ZR, ZR)])

        plsc.subcore_barrier()

        base = sid * CH

        def fetch_idx(off, qs, qd, sem):
            pltpu.async_copy(idx3d.at[0, pl.ds(base + off, QC)], qs, sem)
            pltpu.async_copy(idx3d.at[1, pl.ds(base + off, QC)], qd, sem)

        def wait_idx(off, qs, qd, sem):
            pltpu.make_async_copy(idx3d.at[0, pl.ds(base + off, QC)], qs,
                                  sem).wait()
            pltpu.make_async_copy(idx3d.at[1, pl.ds(base + off, QC)], qd,
                                  sem).wait()

        fetch_idx(0, qs0, qd0, iqsem0)
        wait_idx(0, qs0, qd0, iqsem0)
        pltpu.async_copy(table.at[qs0.at[0]], rows0, gsem0)
        pltpu.async_copy(table.at[qs0.at[1]], rows1, gsem1)
        fetch_idx(QC, qs1, qd1, iqsem1)

        @pl.loop(0, CH, step=2 * QC)
        def _(j):
            for m in range(2 * QC):
                rows_m = rows0 if m % 2 == 0 else rows1
                gsem_m = gsem0 if m % 2 == 0 else gsem1
                qs_m = qs0 if m < QC else qs1
                qd_m = qd0 if m < QC else qd1
                pltpu.make_async_copy(table.at[qs_m.at[m % QC]], rows_m,
                                      gsem_m).wait()
                pltpu.sync_copy(rows_m, acc.at[qd_m.at[m % QC]], add=True)

                if m == QC - 2:
                    wait_idx(j + QC, qs1, qd1, iqsem1)

                if m == QC - 1:
                    @pl.when(j + 2 * QC < CH)
                    def _():
                        fetch_idx(j + 2 * QC, qs0, qd0, iqsem0)

                c = m + 2
                if c < QC:
                    pltpu.async_copy(table.at[qs0.at[c]], rows_m, gsem_m)
                elif c < 2 * QC:
                    pltpu.async_copy(table.at[qs1.at[c - QC]], rows_m,
                                     gsem_m)
                else:
                    if c == 2 * QC:
                        @pl.when(j + 2 * QC < CH)
                        def _():
                            wait_idx(j + 2 * QC, qs0, qd0, iqsem0)

                    @pl.when(j + c < CH)
                    def _():
                        pltpu.async_copy(table.at[qs0.at[c - 2 * QC]],
                                         rows_m, gsem_m)

                if m == 2 * QC - 1:
                    @pl.when(j + 3 * QC < CH)
                    def _():
                        fetch_idx(j + 3 * QC, qs1, qd1, iqsem1)

        plsc.subcore_barrier()
        for k in range(pl.cdiv(NZ, NS)):
            j = sid + k * NS

            @pl.when(j < NZ)
            def _():
                pltpu.sync_copy(acc.at[pl.ds(j * ZR, ZR)],
                                out.at[pl.ds(j * ZR, ZR)])

    @pl.when(cid == 0)
    def _():
        run(tap, idx_ap, out_p)

    @pl.when(cid == 1)
    def _():
        run(tpa, idx_pa, out_a)


def kernel(x_paper, x_author, edge_index_ap, edge_index_pa,
           W_ap_0, W_pa_0, W_sp_0, W_sa_0,
           W_ap_1, W_pa_1, W_sp_1, W_sa_1):
    idx_ap = edge_index_ap.astype(jnp.int32).reshape(2, E // K, K)
    idx_pa = edge_index_pa.astype(jnp.int32).reshape(2, E // K, K)
    zrows_aug = jnp.zeros((ZR, DAUG), jnp.float32)
    zrows = jnp.zeros((ZR, D), jnp.float32)

    sc_agg1 = _make_sc_agg(DAUG)
    sc_agg2 = _make_sc_agg(D)
    sp0, sa0, tap0, tpa0 = _tc1(x_paper, x_author, W_sp_0, W_pa_0, W_sa_0,
                                W_ap_0)
    aggp0, agga0 = sc_agg1(tap0, tpa0, idx_ap, idx_pa, zrows_aug)
    sp1, sa1, tap1, tpa1 = _tc2(aggp0, agga0, sp0, sa0,
                                W_sp_1, W_pa_1, W_sa_1, W_ap_1)
    aggp1, agga1 = sc_agg2(tap1, tpa1, idx_ap, idx_pa, zrows)
    zp, za = _tc3(aggp1, agga1, aggp0, agga0, sp1, sa1)
    return jnp.concatenate([zp, za], axis=0)


# speedup vs baseline: 1.7672x; 1.0024x over previous
import functools

import jax
import jax.numpy as jnp
from jax import lax
from jax.experimental import pallas as pl
from jax.experimental.pallas import tpu as pltpu
from jax.experimental.pallas import tpu_sc as plsc

N = 10000
D = 128
E = 160000
DAUG = 144
K = 125
NS = 16
EPT = E // NS
CH = EPT // K
QC = 8
ZR = 80
NZ = N // ZR
BM = 1000


def _aug_ones(bm):
    return (lax.broadcasted_iota(jnp.int32, (bm, DAUG - D), 1) == 0).astype(
        jnp.float32)


def _tc1_body(xp, xa, wsp, wpa, wsa, wap, sp, sa, tap, tpa):
    xpv = xp[...]
    xav = xa[...]
    sp[...] = jnp.dot(xpv, wsp[...], preferred_element_type=jnp.float32)
    sa[...] = jnp.dot(xav, wsa[...], preferred_element_type=jnp.float32)
    aug = _aug_ones(xpv.shape[0])
    tap[...] = jnp.concatenate(
        [jnp.dot(xav, wap[...], preferred_element_type=jnp.float32), aug],
        axis=1)
    tpa[...] = jnp.concatenate(
        [jnp.dot(xpv, wpa[...], preferred_element_type=jnp.float32), aug],
        axis=1)


def _tc2_body(aggp, agga, sp0, sa0, wsp, wpa, wsa, wap, sp1, sa1, tap, tpa):
    ap = aggp[...]
    aa = agga[...]
    hp = jax.nn.relu(sp0[...] + ap[:, :D] / jnp.clip(ap[:, D:D + 1], 1.0))
    ha = jax.nn.relu(sa0[...] + aa[:, :D] / jnp.clip(aa[:, D:D + 1], 1.0))
    sp1[...] = jnp.dot(hp, wsp[...], preferred_element_type=jnp.float32)
    sa1[...] = jnp.dot(ha, wsa[...], preferred_element_type=jnp.float32)
    tap[...] = jnp.dot(ha, wap[...], preferred_element_type=jnp.float32)
    tpa[...] = jnp.dot(hp, wpa[...], preferred_element_type=jnp.float32)


def _tc3_body(aggp, agga, ag0p, ag0a, sp1, sa1, zp, za):
    zp[...] = sp1[...] + aggp[...] / jnp.clip(ag0p[:, D:D + 1], 1.0)
    za[...] = sa1[...] + agga[...] / jnp.clip(ag0a[:, D:D + 1], 1.0)


_bs_x = pl.BlockSpec((BM, D), lambda i: (i, 0))
_bs_w = pl.BlockSpec((D, D), lambda i: (0, 0))
_bs_aug = pl.BlockSpec((BM, DAUG), lambda i: (i, 0))
_sds_x = jax.ShapeDtypeStruct((N, D), jnp.float32)
_sds_aug = jax.ShapeDtypeStruct((N, DAUG), jnp.float32)

_tc1 = pl.pallas_call(
    _tc1_body,
    grid=(N // BM,),
    in_specs=[_bs_x, _bs_x, _bs_w, _bs_w, _bs_w, _bs_w],
    out_specs=[_bs_x, _bs_x, _bs_aug, _bs_aug],
    out_shape=[_sds_x, _sds_x, _sds_aug, _sds_aug],
)

_tc2 = pl.pallas_call(
    _tc2_body,
    grid=(N // BM,),
    in_specs=[_bs_aug, _bs_aug, _bs_x, _bs_x, _bs_w, _bs_w, _bs_w, _bs_w],
    out_specs=[_bs_x, _bs_x, _bs_x, _bs_x],
    out_shape=[_sds_x, _sds_x, _sds_x, _sds_x],
)

_tc3 = pl.pallas_call(
    _tc3_body,
    grid=(N // BM,),
    in_specs=[_bs_x, _bs_x, _bs_aug, _bs_aug, _bs_x, _bs_x],
    out_specs=[_bs_x, _bs_x],
    out_shape=[_sds_x, _sds_x],
)


@functools.cache
def _make_sc_agg(width):
    mesh = plsc.VectorSubcoreMesh(core_axis_name="c", subcore_axis_name="s")
    return pl.kernel(
        functools.partial(_sc_agg_body, width),
        out_type=[jax.ShapeDtypeStruct((N, width), jnp.float32),
                  jax.ShapeDtypeStruct((N, width), jnp.float32)],
        mesh=mesh,
        scratch_types=[
            pltpu.VMEM((QC, K), jnp.int32),
            pltpu.VMEM((QC, K), jnp.int32),
            pltpu.VMEM((QC, K), jnp.int32),
            pltpu.VMEM((QC, K), jnp.int32),
            pltpu.VMEM((K, width), jnp.float32),
            pltpu.VMEM((K, width), jnp.float32),
            pltpu.VMEM_SHARED((N, width), jnp.float32),
            pltpu.SemaphoreType.DMA,
            pltpu.SemaphoreType.DMA,
            pltpu.SemaphoreType.DMA,
            pltpu.SemaphoreType.DMA,
            pltpu.SemaphoreType.DMA,
        ],
        compiler_params=pltpu.CompilerParams(use_tc_tiling_on_sc=False),
    )


def _sc_agg_body(width, tap, tpa, idx_ap, idx_pa, zrows,
                 out_p, out_a, qs0, qd0, qs1, qd1, rows0, rows1, acc,
                 iqsem0, iqsem1, gsem0, gsem1, zsem):
    cid = lax.axis_index("c")
    sid = lax.axis_index("s")

    def run(table, idx3d, out):
        for k in range(pl.cdiv(NZ, NS)):
            j = sid + k * NS

            @pl.when(j < NZ)
            def _():
                pltpu.async_copy(zrows, acc.at[pl.ds(j * ZR, ZR)], zsem)

        base = sid * CH

        def fetch_idx(off, qs, qd, sem):
            pltpu.async_copy(idx3d.at[0, pl.ds(base + off, QC)], qs, sem)
            pltpu.async_copy(idx3d.at[1, pl.ds(base + off, QC)], qd, sem)

        def wait_idx(off, qs, qd, sem):
            pltpu.make_async_copy(idx3d.at[0, pl.ds(base + off, QC)], qs,
                                  sem).wait()
            pltpu.make_async_copy(idx3d.at[1, pl.ds(base + off, QC)], qd,
                                  sem).wait()

        fetch_idx(0, qs0, qd0, iqsem0)
        wait_idx(0, qs0, qd0, iqsem0)
        pltpu.async_copy(table.at[qs0.at[0]], rows0, gsem0)
        pltpu.async_copy(table.at[qs0.at[1]], rows1, gsem1)
        fetch_idx(QC, qs1, qd1, iqsem1)

        for k in range(pl.cdiv(NZ, NS)):
            j = sid + k * NS

            @pl.when(j < NZ)
            def _():
                pltpu.make_async_copy(zrows, acc.at[pl.ds(j * ZR, ZR)],
                                      zsem).wait()

        plsc.subcore_barrier()

        @pl.loop(0, CH, step=2 * QC)
        def _(j):
            for m in range(2 * QC):
                rows_m = rows0 if m % 2 == 0 else rows1
                gsem_m = gsem0 if m % 2 == 0 else gsem1
                qs_m = qs0 if m < QC else qs1
                qd_m = qd0 if m < QC else qd1
                pltpu.make_async_copy(table.at[qs_m.at[m % QC]], rows_m,
                                      gsem_m).wait()
                pltpu.sync_copy(rows_m, acc.at[qd_m.at[m % QC]], add=True)

                if m == QC - 2:
                    wait_idx(j + QC, qs1, qd1, iqsem1)

                if m == QC - 1:
                    @pl.when(j + 2 * QC < CH)
                    def _():
                        fetch_idx(j + 2 * QC, qs0, qd0, iqsem0)

                c = m + 2
                if c < QC:
                    pltpu.async_copy(table.at[qs0.at[c]], rows_m, gsem_m)
                elif c < 2 * QC:
                    pltpu.async_copy(table.at[qs1.at[c - QC]], rows_m,
                                     gsem_m)
                else:
                    if c == 2 * QC:
                        @pl.when(j + 2 * QC < CH)
                        def _():
                            wait_idx(j + 2 * QC, qs0, qd0, iqsem0)

                    @pl.when(j + c < CH)
                    def _():
                        pltpu.async_copy(table.at[qs0.at[c - 2 * QC]],
                                         rows_m, gsem_m)

                if m == 2 * QC - 1:
                    @pl.when(j + 3 * QC < CH)
                    def _():
                        fetch_idx(j + 3 * QC, qs1, qd1, iqsem1)

        plsc.subcore_barrier()
        for k in range(pl.cdiv(NZ, NS)):
            j = sid + k * NS

            @pl.when(j < NZ)
            def _():
                pltpu.async_copy(acc.at[pl.ds(j * ZR, ZR)],
                                 out.at[pl.ds(j * ZR, ZR)], zsem)
        for k in range(pl.cdiv(NZ, NS)):
            j = sid + k * NS

            @pl.when(j < NZ)
            def _():
                pltpu.make_async_copy(acc.at[pl.ds(j * ZR, ZR)],
                                      out.at[pl.ds(j * ZR, ZR)], zsem).wait()

    @pl.when(cid == 0)
    def _():
        run(tap, idx_ap, out_p)

    @pl.when(cid == 1)
    def _():
        run(tpa, idx_pa, out_a)


def kernel(x_paper, x_author, edge_index_ap, edge_index_pa,
           W_ap_0, W_pa_0, W_sp_0, W_sa_0,
           W_ap_1, W_pa_1, W_sp_1, W_sa_1):
    idx_ap = edge_index_ap.astype(jnp.int32).reshape(2, E // K, K)
    idx_pa = edge_index_pa.astype(jnp.int32).reshape(2, E // K, K)
    zrows_aug = jnp.zeros((ZR, DAUG), jnp.float32)
    zrows = jnp.zeros((ZR, D), jnp.float32)

    sc_agg1 = _make_sc_agg(DAUG)
    sc_agg2 = _make_sc_agg(D)
    sp0, sa0, tap0, tpa0 = _tc1(x_paper, x_author, W_sp_0, W_pa_0, W_sa_0,
                                W_ap_0)
    aggp0, agga0 = sc_agg1(tap0, tpa0, idx_ap, idx_pa, zrows_aug)
    sp1, sa1, tap1, tpa1 = _tc2(aggp0, agga0, sp0, sa0,
                                W_sp_1, W_pa_1, W_sa_1, W_ap_1)
    aggp1, agga1 = sc_agg2(tap1, tpa1, idx_ap, idx_pa, zrows)
    zp, za = _tc3(aggp1, agga1, aggp0, agga0, sp1, sa1)
    return jnp.concatenate([zp, za], axis=0)
